# Initial kernel scaffold; baseline (speedup 1.0000x reference)
import jax
import jax.numpy as jnp
from jax.experimental import pallas as pl


def kernel(inputs, embeddings):
    raise NotImplementedError("write your pallas kernel here")


# speedup vs baseline: 17.5805x; 17.5805x over previous
import functools

import jax
import jax.numpy as jnp
from jax.experimental import pallas as pl

_COMMITMENT_COST = 0.5


def _vq_body(z_ref, emb_ref, q_ref, idx_ref, loss_ref, *, num_heads, head_dim, num_codes):
    z = z_ref[...]
    q_cols = []
    idx_cols = []
    total = jnp.zeros((), jnp.float32)
    for h in range(num_heads):
        zh = z[:, h * head_dim:(h + 1) * head_dim]
        eh = emb_ref[h]
        prod = jnp.dot(zh, eh.T, preferred_element_type=jnp.float32)
        zsq = jnp.sum(zh * zh, axis=1, keepdims=True)
        csq = jnp.sum(eh * eh, axis=1)
        dist = zsq + csq[None, :] - 2.0 * prod
        minv = jnp.min(dist, axis=1, keepdims=True)
        iota = jax.lax.broadcasted_iota(jnp.int32, dist.shape, 1)
        idx = jnp.min(jnp.where(dist == minv, iota, num_codes), axis=1)
        onehot = (iota == idx[:, None]).astype(jnp.float32)
        qh = jnp.dot(onehot, eh, preferred_element_type=jnp.float32)
        q_cols.append(qh)
        idx_cols.append(idx[:, None])
        total = total + jnp.sum(minv)

    q_ref[...] = jnp.concatenate(q_cols, axis=1)
    idx_ref[...] = jnp.concatenate(idx_cols, axis=1)

    total2d = total.reshape(1, 1)

    @pl.when(pl.program_id(0) == 0)
    def _init():
        loss_ref[...] = total2d

    @pl.when(pl.program_id(0) != 0)
    def _acc():
        loss_ref[...] += total2d


def kernel(inputs, embeddings):
    B, T, D = inputs.shape
    H, K, hd = embeddings.shape
    N = B * T
    flat = inputs.reshape(N, D)

    Tb = min(1024, N)
    grid = (N // Tb,)

    body = functools.partial(_vq_body, num_heads=H, head_dim=hd, num_codes=K)
    q, idx, loss_sum = pl.pallas_call(
        body,
        grid=grid,
        in_specs=[
            pl.BlockSpec((Tb, D), lambda i: (i, 0)),
            pl.BlockSpec((H, K, hd), lambda i: (0, 0, 0)),
        ],
        out_specs=[
            pl.BlockSpec((Tb, D), lambda i: (i, 0)),
            pl.BlockSpec((Tb, H), lambda i: (i, 0)),
            pl.BlockSpec((1, 1), lambda i: (0, 0)),
        ],
        out_shape=[
            jax.ShapeDtypeStruct((N, D), jnp.float32),
            jax.ShapeDtypeStruct((N, H), jnp.int32),
            jax.ShapeDtypeStruct((1, 1), jnp.float32),
        ],
    )(flat, embeddings)

    loss = loss_sum[0, 0] * (1.0 + _COMMITMENT_COST) / (N * D)
    return (q.reshape(B, T, D), loss, idx)


# speedup vs baseline: 21.6214x; 1.2298x over previous
import functools

import jax
import jax.numpy as jnp
from jax.experimental import pallas as pl

_COMMITMENT_COST = 0.5


def _vq_body(z_ref, emb_ref, q_ref, idx_ref, loss_ref, *, num_heads, head_dim, num_codes):
    z = z_ref[...]
    q_cols = []
    idx_cols = []
    total = jnp.zeros((), jnp.float32)
    for h in range(num_heads):
        zh = z[:, h * head_dim:(h + 1) * head_dim]
        eh = emb_ref[h]
        prod = jnp.dot(zh, eh.T, preferred_element_type=jnp.float32)
        zsq = jnp.sum(zh * zh, axis=1, keepdims=True)
        csq = jnp.sum(eh * eh, axis=1)
        dist = zsq + csq[None, :] - 2.0 * prod
        minv = jnp.min(dist, axis=1, keepdims=True)
        iota_f = jax.lax.broadcasted_iota(jnp.int32, dist.shape, 1).astype(jnp.float32)
        idx_f = jnp.min(jnp.where(dist == minv, iota_f, float(num_codes)),
                        axis=1, keepdims=True)
        onehot = (iota_f == idx_f).astype(jnp.float32)
        qh = jnp.dot(onehot, eh, preferred_element_type=jnp.float32)
        q_cols.append(qh)
        idx_cols.append(idx_f)
        total = total + jnp.sum(minv)

    q_ref[...] = jnp.concatenate(q_cols, axis=1)
    idx_ref[...] = jnp.concatenate(idx_cols, axis=1).astype(jnp.int32)
    total2d = total.reshape(1, 1)

    @pl.when(pl.program_id(0) == 0)
    def _init():
        loss_ref[...] = total2d

    @pl.when(pl.program_id(0) != 0)
    def _acc():
        loss_ref[...] += total2d


def kernel(inputs, embeddings):
    B, T, D = inputs.shape
    H, K, hd = embeddings.shape
    N = B * T
    flat = inputs.reshape(N, D)

    Tb = min(2048, N)
    grid = (N // Tb,)

    body = functools.partial(_vq_body, num_heads=H, head_dim=hd, num_codes=K)
    q, idx, loss_sum = pl.pallas_call(
        body,
        grid=grid,
        in_specs=[
            pl.BlockSpec((Tb, D), lambda i: (i, 0)),
            pl.BlockSpec((H, K, hd), lambda i: (0, 0, 0)),
        ],
        out_specs=[
            pl.BlockSpec((Tb, D), lambda i: (i, 0)),
            pl.BlockSpec((Tb, H), lambda i: (i, 0)),
            pl.BlockSpec((1, 1), lambda i: (0, 0)),
        ],
        out_shape=[
            jax.ShapeDtypeStruct((N, D), jnp.float32),
            jax.ShapeDtypeStruct((N, H), jnp.int32),
            jax.ShapeDtypeStruct((1, 1), jnp.float32),
        ],
    )(flat, embeddings)

    loss = loss_sum[0, 0] * (1.0 + _COMMITMENT_COST) / (N * D)
    return (q.reshape(B, T, D), loss, idx)
